# baseline (speedup 1.0000x reference)
import jax
import jax.numpy as jnp
from jax.experimental import pallas as pl
from jax.experimental.pallas import tpu as pltpu

NUM_CLASSES = 10000
CLASSES_PER_TASK = 1000
CURRENT_TASK = 1
BAND_START = CURRENT_TASK * CLASSES_PER_TASK
BAND_END = BAND_START + CLASSES_PER_TASK

ROWS_PER_BLOCK = 256


def _band_affine_kernel(alpha_ref, beta_ref, x_ref, o_ref):
    a = alpha_ref[0]
    b = beta_ref[0]
    xv = x_ref[...]
    col = jax.lax.broadcasted_iota(jnp.int32, xv.shape, dimension=1)
    in_band = (col >= BAND_START) & (col < BAND_END)
    o_ref[...] = jnp.where(in_band, a * xv + b, xv)


def kernel(x, alpha, beta):
    m, n = x.shape
    grid = (m // ROWS_PER_BLOCK,)
    return pl.pallas_call(
        _band_affine_kernel,
        grid=grid,
        in_specs=[
            pl.BlockSpec(memory_space=pltpu.SMEM),
            pl.BlockSpec(memory_space=pltpu.SMEM),
            pl.BlockSpec((ROWS_PER_BLOCK, n), lambda i: (i, 0)),
        ],
        out_specs=pl.BlockSpec((ROWS_PER_BLOCK, n), lambda i: (i, 0)),
        out_shape=jax.ShapeDtypeStruct((m, n), x.dtype),
    )(alpha, beta, x)
